# baseline (speedup 1.0000x reference)
import functools
import jax
import jax.numpy as jnp
from jax import lax
from jax.experimental import pallas as pl
from jax.experimental.pallas import tpu as pltpu
from jax.experimental.pallas import tpu_sc as plsc

N_USERS = 100000
N_ITEMS = 100000
K = 64
B = 16384

_SCALE = 4096.0
_INV_SCALE2 = float(2.0 ** -24)
_WROWS = N_USERS // 4

_info = plsc.get_sparse_core_info()
_NC, _NS, _L = _info.num_cores, _info.num_subcores, _info.num_lanes
_NW = _NC * _NS
_BPW = B // _NW
_NG = _BPW // _L
_CH = 256


def _mf_kernel(users_hbm, items_hbm, Uq_hbm, Mq_hbm, Ub_hbm, Mb_hbm, out_hbm,
               idx_uv, idx_iv, u_rows, m_rows, ub_v, mb_v, out_v,
               sem_u, sem_m, sem_b):
    wid = lax.axis_index("s") * _NC + lax.axis_index("c")
    base = wid * _BPW

    pltpu.sync_copy(users_hbm.at[pl.ds(base, _BPW)], idx_uv)
    pltpu.sync_copy(items_hbm.at[pl.ds(base, _BPW)], idx_iv)

    cp_ub = pltpu.async_copy(Ub_hbm.at[idx_uv], ub_v, sem_b)
    cp_mb = pltpu.async_copy(Mb_hbm.at[idx_iv], mb_v, sem_b)

    lane = lax.iota(jnp.int32, _L)

    def dot16(row_ref, j, quarter):
        off = quarter * 32
        parts = []
        for t in range(2):
            w = row_ref[j, pl.ds(off + t * _L, _L)]
            lo = jax.lax.shift_right_arithmetic(
                jax.lax.shift_left(w, 16), 16)
            hi = jax.lax.shift_right_arithmetic(w, 16)
            parts.append(lo.astype(jnp.float32))
            parts.append(hi.astype(jnp.float32))
        return parts

    def chunk_body(c, _):
        cb = c * _CH

        def fire(g):
            gb = g * _L
            ru = jax.lax.shift_right_logical(idx_uv[pl.ds(cb + gb, _L)], 2)
            ri = jax.lax.shift_right_logical(idx_iv[pl.ds(cb + gb, _L)], 2)
            for r in range(_L):
                j = gb + r
                pltpu.async_copy(Uq_hbm.at[pl.ds(ru[r], 1), :],
                                 u_rows.at[pl.ds(j, 1), :], sem_u)
                pltpu.async_copy(Mq_hbm.at[pl.ds(ri[r], 1), :],
                                 m_rows.at[pl.ds(j, 1), :], sem_m)

        def drain(g):
            gb = g * _L
            for r in range(_L):
                j = gb + r
                pltpu.make_async_copy(
                    Uq_hbm.at[pl.ds(0, 1), :],
                    u_rows.at[pl.ds(j, 1), :], sem_u).wait()
                pltpu.make_async_copy(
                    Mq_hbm.at[pl.ds(0, 1), :],
                    m_rows.at[pl.ds(j, 1), :], sem_m).wait()

        fire(0)

        def dma_body(g, _):
            fire(g)
            drain(g - 1)
            return 0

        lax.fori_loop(1, _CH // _L, dma_body, 0)
        drain(_CH // _L - 1)

        def group_body(g, _):
            gb = g * _L
            qu = idx_uv[pl.ds(cb + gb, _L)] & 3
            qi = idx_iv[pl.ds(cb + gb, _L)] & 3
            res = jnp.zeros((_L,), jnp.float32)
            for r in range(_L):
                j = gb + r
                us = dot16(u_rows, j, qu[r])
                ms = dot16(m_rows, j, qi[r])
                acc = None
                for t in range(4):
                    p = us[t] * ms[t]
                    acc = p if acc is None else acc + p
                s = jnp.sum(acc) * _INV_SCALE2
                res = jnp.where(lane == r, s, res)
            res = res + ub_v[pl.ds(cb + gb, _L)] + mb_v[pl.ds(cb + gb, _L)]
            out_v[pl.ds(cb + gb, _L)] = res
            return 0

        lax.fori_loop(0, _CH // _L, group_body, 0)
        return 0

    cp_ub.wait()
    cp_mb.wait()
    lax.fori_loop(0, _BPW // _CH, chunk_body, 0)

    pltpu.sync_copy(out_v, out_hbm.at[pl.ds(base, _BPW)])


@jax.jit
def _run(users, items, Uq, Mq, Ub, Mb):
    mesh = plsc.VectorSubcoreMesh(core_axis_name="c", subcore_axis_name="s")
    kfn = functools.partial(
        pl.kernel,
        out_type=jax.ShapeDtypeStruct((B,), jnp.float32),
        mesh=mesh,
        scratch_types=[
            pltpu.VMEM((_BPW,), jnp.int32),
            pltpu.VMEM((_BPW,), jnp.int32),
            pltpu.VMEM((_CH, 128), jnp.int32),
            pltpu.VMEM((_CH, 128), jnp.int32),
            pltpu.VMEM((_BPW,), jnp.float32),
            pltpu.VMEM((_BPW,), jnp.float32),
            pltpu.VMEM((_BPW,), jnp.float32),
            pltpu.SemaphoreType.DMA,
            pltpu.SemaphoreType.DMA,
            pltpu.SemaphoreType.DMA,
        ],
        compiler_params=pltpu.CompilerParams(needs_layout_passes=False),
    )(_mf_kernel)
    return kfn(users, items, Uq, Mq, Ub, Mb)


def _quantize_pack(T):
    q = jnp.clip(jnp.round(T * _SCALE), -32767.0, 32767.0).astype(jnp.int32)
    lo = q[:, 0::2] & 0xFFFF
    hi = jax.lax.shift_left(q[:, 1::2], 16)
    return (lo | hi).reshape(_WROWS, 128)


def kernel(users, items, movie_map, U, M, Ub, Mb):
    del movie_map
    return _run(users.astype(jnp.int32), items.astype(jnp.int32),
                _quantize_pack(U), _quantize_pack(M),
                Ub.reshape(-1), Mb.reshape(-1))


# speedup vs baseline: 4.0416x; 4.0416x over previous
import functools
import jax
import jax.numpy as jnp
from jax import lax
from jax.experimental import pallas as pl
from jax.experimental.pallas import tpu as pltpu
from jax.experimental.pallas import tpu_sc as plsc

N_USERS = 100000
N_ITEMS = 100000
K = 64
B = 16384

_info = plsc.get_sparse_core_info()
_NC, _NS, _L = _info.num_cores, _info.num_subcores, _info.num_lanes
_NW = _NC * _NS
_BPW = B // _NW
_CH = 128
_NCH = _BPW // _CH


def _mf_kernel(users_hbm, items_hbm, U_hbm, M_hbm, Ub_hbm, Mb_hbm, out_hbm,
               idx_uv, idx_iv, u_rows, m_rows, ub_v, mb_v, out_v,
               sem_u, sem_m, sem_b):
    wid = lax.axis_index("s") * _NC + lax.axis_index("c")
    base = wid * _BPW

    pltpu.sync_copy(users_hbm.at[pl.ds(base, _BPW)], idx_uv)
    pltpu.sync_copy(items_hbm.at[pl.ds(base, _BPW)], idx_iv)

    cp_ub = pltpu.async_copy(Ub_hbm.at[idx_uv], ub_v, sem_b)
    cp_mb = pltpu.async_copy(Mb_hbm.at[idx_iv], mb_v, sem_b)

    lane = lax.iota(jnp.int32, _L)

    def fire(c):
        cb = c * _CH

        def fire_g(g, _):
            gb = g * _L
            ru = idx_uv[pl.ds(cb + gb, _L)]
            ri = idx_iv[pl.ds(cb + gb, _L)]
            for r in range(_L):
                j = gb + r
                pltpu.async_copy(U_hbm.at[pl.ds(ru[r], 1), :],
                                 u_rows.at[c % 2, pl.ds(j, 1), :], sem_u)
                pltpu.async_copy(M_hbm.at[pl.ds(ri[r], 1), :],
                                 m_rows.at[c % 2, pl.ds(j, 1), :], sem_m)
            return 0

        lax.fori_loop(0, _CH // _L, fire_g, 0)

    def drain(c):
        def drain_g(g, _):
            gb = g * _L
            for r in range(_L):
                j = gb + r
                pltpu.make_async_copy(
                    U_hbm.at[pl.ds(0, 1), :],
                    u_rows.at[c % 2, pl.ds(j, 1), :], sem_u).wait()
                pltpu.make_async_copy(
                    M_hbm.at[pl.ds(0, 1), :],
                    m_rows.at[c % 2, pl.ds(j, 1), :], sem_m).wait()
            return 0

        lax.fori_loop(0, _CH // _L, drain_g, 0)

    def compute(c):
        cb = c * _CH
        ur = u_rows.at[c % 2]
        mr = m_rows.at[c % 2]

        def group_body(g, _):
            gb = g * _L
            res = jnp.zeros((_L,), jnp.float32)
            for r in range(_L):
                j = gb + r
                acc = None
                for t in range(K // _L):
                    u = ur[j, pl.ds(t * _L, _L)]
                    m = mr[j, pl.ds(t * _L, _L)]
                    p = u * m
                    acc = p if acc is None else acc + p
                s = jnp.sum(acc)
                res = jnp.where(lane == r, s, res)
            res = res + ub_v[pl.ds(cb + gb, _L)] + mb_v[pl.ds(cb + gb, _L)]
            out_v[pl.ds(cb + gb, _L)] = res
            return 0

        lax.fori_loop(0, _CH // _L, group_body, 0)

    cp_ub.wait()
    cp_mb.wait()
    fire(0)
    for c in range(_NCH):
        drain(c)
        if c + 1 < _NCH:
            fire(c + 1)
        compute(c)

    pltpu.sync_copy(out_v, out_hbm.at[pl.ds(base, _BPW)])


@jax.jit
def _run(users, items, U, M, Ub, Mb):
    mesh = plsc.VectorSubcoreMesh(core_axis_name="c", subcore_axis_name="s")
    kfn = functools.partial(
        pl.kernel,
        out_type=jax.ShapeDtypeStruct((B,), jnp.float32),
        mesh=mesh,
        scratch_types=[
            pltpu.VMEM((_BPW,), jnp.int32),
            pltpu.VMEM((_BPW,), jnp.int32),
            pltpu.VMEM((2, _CH, K), jnp.float32),
            pltpu.VMEM((2, _CH, K), jnp.float32),
            pltpu.VMEM((_BPW,), jnp.float32),
            pltpu.VMEM((_BPW,), jnp.float32),
            pltpu.VMEM((_BPW,), jnp.float32),
            pltpu.SemaphoreType.DMA,
            pltpu.SemaphoreType.DMA,
            pltpu.SemaphoreType.DMA,
        ],
        compiler_params=pltpu.CompilerParams(needs_layout_passes=False),
    )(_mf_kernel)
    return kfn(users, items, U, M, Ub, Mb)


def kernel(users, items, movie_map, U, M, Ub, Mb):
    del movie_map
    return _run(users.astype(jnp.int32), items.astype(jnp.int32),
                U, M, Ub.reshape(-1), Mb.reshape(-1))


# speedup vs baseline: 4.1121x; 1.0174x over previous
import functools
import jax
import jax.numpy as jnp
from jax import lax
from jax.experimental import pallas as pl
from jax.experimental.pallas import tpu as pltpu
from jax.experimental.pallas import tpu_sc as plsc

N_USERS = 100000
N_ITEMS = 100000
K = 64
B = 16384

_info = plsc.get_sparse_core_info()
_NC, _NS, _L = _info.num_cores, _info.num_subcores, _info.num_lanes
_NW = _NC * _NS
_BPW = B // _NW
_CH = 128
_NCH = _BPW // _CH


def _mf_kernel(users_hbm, items_hbm, U_hbm, M_hbm, Ub_hbm, Mb_hbm, out_hbm,
               idx_uv, idx_iv, u_rows, m_rows, ub_v, mb_v, out_v,
               sem_u, sem_m, sem_b):
    wid = lax.axis_index("s") * _NC + lax.axis_index("c")
    base = wid * _BPW

    pltpu.sync_copy(users_hbm.at[pl.ds(base, _BPW)], idx_uv)
    pltpu.sync_copy(items_hbm.at[pl.ds(base, _BPW)], idx_iv)

    cp_ub = pltpu.async_copy(Ub_hbm.at[idx_uv], ub_v, sem_b)
    cp_mb = pltpu.async_copy(Mb_hbm.at[idx_iv], mb_v, sem_b)

    lane = lax.iota(jnp.int32, _L)

    def fire(c):
        cb = c * _CH

        def fire_g(g, _):
            gb = g * _L
            ru = idx_uv[pl.ds(cb + gb, _L)]
            ri = idx_iv[pl.ds(cb + gb, _L)]
            for r in range(_L):
                j = gb + r
                pltpu.async_copy(U_hbm.at[pl.ds(ru[r], 1), :],
                                 u_rows.at[c % 2, pl.ds(j, 1), :], sem_u)
                pltpu.async_copy(M_hbm.at[pl.ds(ri[r], 1), :],
                                 m_rows.at[c % 2, pl.ds(j, 1), :], sem_m)
            return 0

        lax.fori_loop(0, _CH // _L, fire_g, 0)

    def drain(c):
        pltpu.make_async_copy(U_hbm.at[pl.ds(0, _CH), :],
                              u_rows.at[c % 2], sem_u).wait()
        pltpu.make_async_copy(M_hbm.at[pl.ds(0, _CH), :],
                              m_rows.at[c % 2], sem_m).wait()

    def compute(c):
        cb = c * _CH
        ur = u_rows.at[c % 2]
        mr = m_rows.at[c % 2]

        def group_body(g, _):
            gb = g * _L
            res = jnp.zeros((_L,), jnp.float32)
            for r in range(_L):
                j = gb + r
                acc = None
                for t in range(K // _L):
                    u = ur[j, pl.ds(t * _L, _L)]
                    m = mr[j, pl.ds(t * _L, _L)]
                    p = u * m
                    acc = p if acc is None else acc + p
                s = jnp.sum(acc)
                res = jnp.where(lane == r, s, res)
            res = res + ub_v[pl.ds(cb + gb, _L)] + mb_v[pl.ds(cb + gb, _L)]
            out_v[pl.ds(cb + gb, _L)] = res
            return 0

        lax.fori_loop(0, _CH // _L, group_body, 0)

    cp_ub.wait()
    cp_mb.wait()
    fire(0)
    for c in range(_NCH):
        drain(c)
        if c + 1 < _NCH:
            fire(c + 1)
        compute(c)

    pltpu.sync_copy(out_v, out_hbm.at[pl.ds(base, _BPW)])


@jax.jit
def _run(users, items, U, M, Ub, Mb):
    mesh = plsc.VectorSubcoreMesh(core_axis_name="c", subcore_axis_name="s")
    kfn = functools.partial(
        pl.kernel,
        out_type=jax.ShapeDtypeStruct((B,), jnp.float32),
        mesh=mesh,
        scratch_types=[
            pltpu.VMEM((_BPW,), jnp.int32),
            pltpu.VMEM((_BPW,), jnp.int32),
            pltpu.VMEM((2, _CH, K), jnp.float32),
            pltpu.VMEM((2, _CH, K), jnp.float32),
            pltpu.VMEM((_BPW,), jnp.float32),
            pltpu.VMEM((_BPW,), jnp.float32),
            pltpu.VMEM((_BPW,), jnp.float32),
            pltpu.SemaphoreType.DMA,
            pltpu.SemaphoreType.DMA,
            pltpu.SemaphoreType.DMA,
        ],
        compiler_params=pltpu.CompilerParams(needs_layout_passes=False),
    )(_mf_kernel)
    return kfn(users, items, U, M, Ub, Mb)


def kernel(users, items, movie_map, U, M, Ub, Mb):
    del movie_map
    return _run(users.astype(jnp.int32), items.astype(jnp.int32),
                U, M, Ub.reshape(-1), Mb.reshape(-1))


# speedup vs baseline: 4.1630x; 1.0124x over previous
import functools
import jax
import jax.numpy as jnp
from jax import lax
from jax.experimental import pallas as pl
from jax.experimental.pallas import tpu as pltpu
from jax.experimental.pallas import tpu_sc as plsc

N_USERS = 100000
N_ITEMS = 100000
K = 64
B = 16384

_info = plsc.get_sparse_core_info()
_NC, _NS, _L = _info.num_cores, _info.num_subcores, _info.num_lanes
_NW = _NC * _NS
_BPW = B // _NW
_CH = 128
_NCH = _BPW // _CH


def _mf_kernel(users_hbm, items_hbm, U_hbm, M_hbm, Ub_hbm, Mb_hbm, out_hbm,
               idx_uv, idx_iv, u_rows, m_rows, ub_v, mb_v, out_v,
               sem_u0, sem_u1, sem_m0, sem_m1, sem_b):
    wid = lax.axis_index("s") * _NC + lax.axis_index("c")
    base = wid * _BPW

    pltpu.sync_copy(users_hbm.at[pl.ds(base, _BPW)], idx_uv)
    pltpu.sync_copy(items_hbm.at[pl.ds(base, _BPW)], idx_iv)

    cp_ub = pltpu.async_copy(Ub_hbm.at[idx_uv], ub_v, sem_b)
    cp_mb = pltpu.async_copy(Mb_hbm.at[idx_iv], mb_v, sem_b)

    lane = lax.iota(jnp.int32, _L)

    def fire(c):
        cb = c * _CH

        def fire_g(g, _):
            gb = g * _L
            ru = idx_uv[pl.ds(cb + gb, _L)]
            ri = idx_iv[pl.ds(cb + gb, _L)]
            for r in range(_L):
                j = gb + r
                pltpu.async_copy(U_hbm.at[pl.ds(ru[r], 1), :],
                                 u_rows.at[c % 2, pl.ds(j, 1), :],
                                 sem_u0 if c % 2 == 0 else sem_u1)
                pltpu.async_copy(M_hbm.at[pl.ds(ri[r], 1), :],
                                 m_rows.at[c % 2, pl.ds(j, 1), :],
                                 sem_m0 if c % 2 == 0 else sem_m1)
            return 0

        lax.fori_loop(0, _CH // _L, fire_g, 0)

    def drain(c):
        pltpu.make_async_copy(U_hbm.at[pl.ds(0, _CH), :],
                              u_rows.at[c % 2],
                              sem_u0 if c % 2 == 0 else sem_u1).wait()
        pltpu.make_async_copy(M_hbm.at[pl.ds(0, _CH), :],
                              m_rows.at[c % 2],
                              sem_m0 if c % 2 == 0 else sem_m1).wait()

    def compute(c):
        cb = c * _CH
        ur = u_rows.at[c % 2]
        mr = m_rows.at[c % 2]

        def group_body(g, _):
            gb = g * _L
            res = jnp.zeros((_L,), jnp.float32)
            for r in range(_L):
                j = gb + r
                acc = None
                for t in range(K // _L):
                    u = ur[j, pl.ds(t * _L, _L)]
                    m = mr[j, pl.ds(t * _L, _L)]
                    p = u * m
                    acc = p if acc is None else acc + p
                s = jnp.sum(acc)
                res = jnp.where(lane == r, s, res)
            res = res + ub_v[pl.ds(cb + gb, _L)] + mb_v[pl.ds(cb + gb, _L)]
            out_v[pl.ds(cb + gb, _L)] = res
            return 0

        lax.fori_loop(0, _CH // _L, group_body, 0)

    cp_ub.wait()
    cp_mb.wait()
    fire(0)
    for c in range(_NCH):
        if c + 1 < _NCH:
            fire(c + 1)
        drain(c)
        compute(c)

    pltpu.sync_copy(out_v, out_hbm.at[pl.ds(base, _BPW)])


@jax.jit
def _run(users, items, U, M, Ub, Mb):
    mesh = plsc.VectorSubcoreMesh(core_axis_name="c", subcore_axis_name="s")
    kfn = functools.partial(
        pl.kernel,
        out_type=jax.ShapeDtypeStruct((B,), jnp.float32),
        mesh=mesh,
        scratch_types=[
            pltpu.VMEM((_BPW,), jnp.int32),
            pltpu.VMEM((_BPW,), jnp.int32),
            pltpu.VMEM((2, _CH, K), jnp.float32),
            pltpu.VMEM((2, _CH, K), jnp.float32),
            pltpu.VMEM((_BPW,), jnp.float32),
            pltpu.VMEM((_BPW,), jnp.float32),
            pltpu.VMEM((_BPW,), jnp.float32),
            pltpu.SemaphoreType.DMA,
            pltpu.SemaphoreType.DMA,
            pltpu.SemaphoreType.DMA,
            pltpu.SemaphoreType.DMA,
            pltpu.SemaphoreType.DMA,
        ],
        compiler_params=pltpu.CompilerParams(needs_layout_passes=False),
    )(_mf_kernel)
    return kfn(users, items, U, M, Ub, Mb)


def kernel(users, items, movie_map, U, M, Ub, Mb):
    del movie_map
    return _run(users.astype(jnp.int32), items.astype(jnp.int32),
                U, M, Ub.reshape(-1), Mb.reshape(-1))
